# Initial kernel scaffold; baseline (speedup 1.0000x reference)
import jax
import jax.numpy as jnp
from jax.experimental import pallas as pl


def kernel(x, table, W_proj, b_proj, W1, b1, W2, b2):
    raise NotImplementedError("write your pallas kernel here")


# speedup vs baseline: 8.2089x; 8.2089x over previous
import jax
import jax.numpy as jnp
from jax import lax
from jax.experimental import pallas as pl
from jax.experimental.pallas import tpu as pltpu
from jax.experimental.pallas import tpu_sc as plsc

_B, _T, _V, _NQ = 1024, 50, 22, 2
_K, _D, _H, _C = 8192, 16, 128, 60
_NIDX = _T * _V * _NQ

_NC, _NS, _L = 2, 16, 16
_NW = _NC * _NS
_BPW = _B // _NW
_NCHUNK = _NIDX // _L
_TAIL = _NIDX - _NCHUNK * _L
_IDXBUF = _NCHUNK * _L + _L


def _sc_hist_body(x_ref, counts_ref, idx_v, bins):
    wid = lax.axis_index("s") * _NC + lax.axis_index("c")
    base = wid * _BPW

    zeros16 = jnp.zeros((_L,), jnp.float32)
    ones16 = jnp.ones((_L,), jnp.float32)
    tail_mask = lax.iota(jnp.int32, _L) < _TAIL

    def zero_body(i, c):
        bins[pl.ds(i * _L, _L)] = zeros16
        return c
    lax.fori_loop(0, _K // _L, zero_body, 0)

    def batch_body(bl, c):
        b = base + bl
        pltpu.sync_copy(x_ref.at[pl.ds(b * _NIDX, _NIDX)],
                        idx_v.at[pl.ds(0, _NIDX)])

        def add_body(i, c2):
            idx = idx_v[pl.ds(i * _L, _L)]
            plsc.addupdate_scatter(bins, [idx], ones16)
            return c2
        lax.fori_loop(0, _NCHUNK, add_body, 0)
        t_idx = idx_v[pl.ds(_NCHUNK * _L, _L)]
        plsc.addupdate_scatter(bins, [t_idx], ones16, mask=tail_mask)

        pltpu.sync_copy(bins, counts_ref.at[b])

        def z_body(i, c2):
            idx = idx_v[pl.ds(i * _L, _L)]
            plsc.store_scatter(bins, [idx], zeros16)
            return c2
        lax.fori_loop(0, _NCHUNK, z_body, 0)
        plsc.store_scatter(bins, [t_idx], zeros16, mask=tail_mask)
        return c

    lax.fori_loop(0, _BPW, batch_body, 0)


def _sc_histogram(x2d):
    mesh = plsc.VectorSubcoreMesh(core_axis_name="c", subcore_axis_name="s")
    return pl.kernel(
        _sc_hist_body,
        out_type=jax.ShapeDtypeStruct((_B, _K), jnp.float32),
        mesh=mesh,
        compiler_params=pltpu.CompilerParams(needs_layout_passes=False),
        scratch_types=[
            pltpu.VMEM((_IDXBUF,), jnp.int32),
            pltpu.VMEM((_K,), jnp.float32),
        ],
    )(x2d)


_BB = 128


def _tc_mlp_body(counts_ref, table_ref, wp_ref, bp_ref, w1_ref, b1_ref,
                 w2_ref, b2_ref, out_ref):
    c = counts_ref[...]
    pooled = jnp.dot(c, table_ref[...],
                     preferred_element_type=jnp.float32) * (1.0 / _NIDX)
    feat = jnp.dot(pooled, wp_ref[...],
                   preferred_element_type=jnp.float32) + bp_ref[...]
    h = jnp.maximum(
        jnp.dot(feat, w1_ref[...], preferred_element_type=jnp.float32)
        + b1_ref[...], 0.0)
    out_ref[...] = jnp.dot(h, w2_ref[...],
                           preferred_element_type=jnp.float32) + b2_ref[...]


def _tc_mlp(counts, table, W_proj, b_proj, W1, b1, W2, b2):
    full = lambda shape: pl.BlockSpec(shape, lambda i: (0, 0))
    return pl.pallas_call(
        _tc_mlp_body,
        grid=(_B // _BB,),
        in_specs=[
            pl.BlockSpec((_BB, _K), lambda i: (i, 0)),
            full((_K, _D)),
            full((_D, _D)),
            full((1, _D)),
            full((_D, _H)),
            full((1, _H)),
            full((_H, _C)),
            full((1, _C)),
        ],
        out_specs=pl.BlockSpec((_BB, _C), lambda i: (i, 0)),
        out_shape=jax.ShapeDtypeStruct((_B, _C), jnp.float32),
    )(counts, table, W_proj, b_proj.reshape(1, _D), W1, b1.reshape(1, _H),
      W2, b2.reshape(1, _C))


def kernel(x, table, W_proj, b_proj, W1, b1, W2, b2):
    x2d = x.reshape(_B * _NIDX)
    counts = _sc_histogram(x2d)
    return _tc_mlp(counts, table, W_proj, b_proj, W1, b1, W2, b2)
